# baseline (speedup 1.0000x reference)
import functools

import jax
import jax.numpy as jnp
from jax import lax
from jax.experimental import pallas as pl
from jax.experimental.pallas import tpu as pltpu
from jax.experimental.pallas import tpu_sc as plsc

_V = 50000
_H = 768
_L = 16
_HC = _H // _L

_NC = 2
_NS = 16
_NW = _NC * _NS


def _rsqrt(x):
    i = lax.bitcast_convert_type(x, jnp.int32)
    i = jnp.int32(0x5F3759DF) - lax.shift_right_arithmetic(i, 1)
    y = lax.bitcast_convert_type(i, jnp.float32)
    for _ in range(3):
        y = y * (1.5 - 0.5 * x * y * y)
    return y


def _lane_sum(x):
    idx = lax.iota(jnp.int32, _L)
    for k in (8, 4, 2, 1):
        x = x + x.at[jnp.bitwise_xor(idx, k)].get(mode="promise_in_bounds")
    return x


def _make_emb_kernel(n_tok, seq_len, chunk):
    tpw = n_tok // _NW
    n_chunks = tpw // chunk
    mesh = plsc.VectorSubcoreMesh(core_axis_name="c", subcore_axis_name="s")

    @functools.partial(
        pl.kernel,
        out_type=jax.ShapeDtypeStruct((n_tok, _H), jnp.float32),
        mesh=mesh,
        scratch_types=[
            pltpu.VMEM((tpw,), jnp.int32),
            pltpu.VMEM((tpw + _L,), jnp.int32),
            pltpu.VMEM((chunk, _H), jnp.float32),
            pltpu.VMEM((chunk, _H), jnp.float32),
            pltpu.VMEM((2, _H), jnp.float32),
            pltpu.VMEM((_H,), jnp.float32),
            pltpu.VMEM((_H,), jnp.float32),
            pltpu.SemaphoreType.DMA,
        ],
    )
    def emb_kernel(ids_hbm, tt_hbm, word_hbm, pos_hbm, type_hbm,
                   lnw_hbm, lnb_hbm, out_hbm,
                   idx_v, ttv, rows_v, pos_v, ttab_v, lnw_v, lnb_v, sem):
        wid = lax.axis_index("s") * _NC + lax.axis_index("c")
        base = wid * tpw
        pos_base = lax.rem(base, seq_len)

        pltpu.sync_copy(ids_hbm.at[pl.ds(base, tpw)], idx_v)
        pltpu.sync_copy(tt_hbm.at[pl.ds(base, tpw)], ttv.at[pl.ds(0, tpw)])
        pltpu.sync_copy(type_hbm, ttab_v)
        pltpu.sync_copy(lnw_hbm, lnw_v)
        pltpu.sync_copy(lnb_hbm, lnb_v)

        def chunk_body(ch, carry):
            off = ch * chunk
            pltpu.async_copy(
                word_hbm.at[idx_v.at[pl.ds(off, chunk)]], rows_v, sem
            ).wait()
            pltpu.sync_copy(pos_hbm.at[pl.ds(pos_base + off, chunk)], pos_v)

            def tok_body(t, tc):
                tt = ttv[pl.ds(off + t, _L)][0]
                s = jnp.zeros((_L,), jnp.float32)
                q = jnp.zeros((_L,), jnp.float32)
                for c in range(_HC):
                    sl = pl.ds(c * _L, _L)
                    v = rows_v[t, sl] + pos_v[t, sl] + ttab_v[tt, sl]
                    rows_v[t, sl] = v
                    s = s + v
                    q = q + v * v
                tot = _lane_sum(s)
                tot2 = _lane_sum(q)
                mean = tot * (1.0 / _H)
                var = tot2 * (1.0 / _H) - mean * mean
                r = _rsqrt(var + 1e-5)
                for c in range(_HC):
                    sl = pl.ds(c * _L, _L)
                    rows_v[t, sl] = ((rows_v[t, sl] - mean) * (lnw_v[sl] * r)
                                     + lnb_v[sl])
                return tc

            lax.fori_loop(0, chunk, tok_body, 0)
            pltpu.sync_copy(rows_v, out_hbm.at[pl.ds(base + off, chunk)])
            return carry

        lax.fori_loop(0, n_chunks, chunk_body, 0)

    return emb_kernel


def kernel(input_ids, token_type_ids, word_emb, pos_emb, type_emb, ln_w, ln_b):
    b, s = input_ids.shape
    n_tok = b * s
    ids = input_ids.reshape(n_tok).astype(jnp.int32)
    tts = token_type_ids.reshape(n_tok).astype(jnp.int32)
    emb = _make_emb_kernel(n_tok, s, chunk=32)
    out = emb(ids, tts, word_emb, pos_emb, type_emb, ln_w, ln_b)
    return out.reshape(b, s, _H)
